# baseline (speedup 1.0000x reference)
import functools

import jax
import jax.numpy as jnp
from jax import lax
from jax.experimental import pallas as pl
from jax.experimental.pallas import tpu as pltpu
from jax.experimental.pallas import tpu_sc as plsc

_N = 10000
_E = 160000
_DIN = 128
_NW = 32
_PER_W = _E // _NW
_CH = 128
_NFULL = _PER_W // _CH
_TAIL = _PER_W - _NFULL * _CH
_ZROWS = 625


def _sc_gather(table, idx, d):
    dt = table.dtype
    mesh = plsc.VectorSubcoreMesh(core_axis_name="c", subcore_axis_name="s")

    @functools.partial(
        pl.kernel,
        out_type=jax.ShapeDtypeStruct((_E, d), dt),
        mesh=mesh,
        compiler_params=pltpu.CompilerParams(use_tc_tiling_on_sc=False),
        scratch_types=[
            pltpu.VMEM((_CH,), jnp.int32),
            pltpu.VMEM((_CH, d), dt),
            pltpu.VMEM((_TAIL,), jnp.int32),
            pltpu.VMEM((_TAIL, d), dt),
            pltpu.SemaphoreType.DMA,
        ],
    )
    def k(table_h, idx_h, out_h, idx_v, rows_v, idx_t, rows_t, sem):
        wid = lax.axis_index("s") * 2 + lax.axis_index("c")
        base = wid * _PER_W

        def body(j, carry):
            off = base + j * _CH
            pltpu.sync_copy(idx_h.at[pl.ds(off, _CH)], idx_v)
            pltpu.async_copy(table_h.at[idx_v], rows_v, sem).wait()
            pltpu.sync_copy(rows_v, out_h.at[pl.ds(off, _CH)])
            return carry

        lax.fori_loop(0, _NFULL, body, 0)
        off = base + _NFULL * _CH
        pltpu.sync_copy(idx_h.at[pl.ds(off, _TAIL)], idx_t)
        pltpu.async_copy(table_h.at[idx_t], rows_t, sem).wait()
        pltpu.sync_copy(rows_t, out_h.at[pl.ds(off, _TAIL)])

    return k(table, idx)


def _sc_scatter(msg, dst, zrows):
    mesh = plsc.VectorSubcoreMesh(core_axis_name="c", subcore_axis_name="s")

    @functools.partial(
        pl.kernel,
        out_type=jax.ShapeDtypeStruct((2, _N, 16), jnp.float32),
        mesh=mesh,
        compiler_params=pltpu.CompilerParams(use_tc_tiling_on_sc=False),
        scratch_types=[
            pltpu.VMEM((_CH,), jnp.int32),
            pltpu.VMEM((_CH, 16), jnp.float32),
            pltpu.VMEM((_TAIL,), jnp.int32),
            pltpu.VMEM((_TAIL, 16), jnp.float32),
            pltpu.VMEM_SHARED((_N, 16), jnp.float32),
        ],
    )
    def k(msg_h, dst_h, z_h, out_h, idx_v, msg_v, idx_t, msg_t, agg):
        cid = lax.axis_index("c")
        sid = lax.axis_index("s")
        pltpu.sync_copy(z_h, agg.at[pl.ds(sid * _ZROWS, _ZROWS)])
        plsc.subcore_barrier()

        wid = sid * 2 + cid
        base = wid * _PER_W

        def body(j, carry):
            off = base + j * _CH
            pltpu.sync_copy(dst_h.at[pl.ds(off, _CH)], idx_v)
            pltpu.sync_copy(msg_h.at[pl.ds(off, _CH)], msg_v)
            pltpu.sync_copy(msg_v, agg.at[idx_v], add=True)
            return carry

        lax.fori_loop(0, _NFULL, body, 0)
        off = base + _NFULL * _CH
        pltpu.sync_copy(dst_h.at[pl.ds(off, _TAIL)], idx_t)
        pltpu.sync_copy(msg_h.at[pl.ds(off, _TAIL)], msg_t)
        pltpu.sync_copy(msg_t, agg.at[idx_t], add=True)

        plsc.subcore_barrier()

        @pl.when(sid == 0)
        def _():
            pltpu.sync_copy(agg, out_h.at[cid])

    return k(msg, dst, zrows)


def _edge1_body(ea_ref, xj_ref, w1r_ref, b1r_ref, a1_ref, b2r_ref, s1_ref,
                out_ref):
    ea = ea_ref[...]
    hrep = jnp.maximum(ea * w1r_ref[...] + b1r_ref[...], 0.0)
    g = jnp.dot(xj_ref[...], a1_ref[...],
                preferred_element_type=jnp.float32)
    prod = hrep * g
    msg = jnp.dot(prod, s1_ref[...], preferred_element_type=jnp.float32)
    msg = msg + jnp.dot(xj_ref[...], b2r_ref[...],
                        preferred_element_type=jnp.float32)
    k = msg.shape[0]
    col = lax.broadcasted_iota(jnp.int32, (k, 16), 1)
    out_ref[...] = msg + (col == 8).astype(jnp.float32)


def _tc_edge1(ea, xj, w1rep, b1rep, a1, b2r, s1):
    kk = 2000
    grid = _E // kk
    return pl.pallas_call(
        _edge1_body,
        grid=(grid,),
        in_specs=[
            pl.BlockSpec((kk, 1), lambda i: (i, 0)),
            pl.BlockSpec((kk, _DIN), lambda i: (i, 0)),
            pl.BlockSpec((1, 512), lambda i: (0, 0)),
            pl.BlockSpec((1, 512), lambda i: (0, 0)),
            pl.BlockSpec((_DIN, 512), lambda i: (0, 0)),
            pl.BlockSpec((_DIN, 16), lambda i: (0, 0)),
            pl.BlockSpec((512, 16), lambda i: (0, 0)),
        ],
        out_specs=pl.BlockSpec((kk, 16), lambda i: (i, 0)),
        out_shape=jax.ShapeDtypeStruct((_E, 16), jnp.float32),
    )(ea, xj, w1rep, b1rep, a1, b2r, s1)


def _edge2_body(ea_ref, hj_ref, w1_ref, b1_ref, w2p_ref, b2p_ref, r2_ref,
                s2_ref, out_ref):
    ea = ea_ref[...]
    h2 = jnp.maximum(ea * w1_ref[...] + b1_ref[...], 0.0)
    w2p = jnp.dot(h2, w2p_ref[...],
                  preferred_element_type=jnp.float32) + b2p_ref[...]
    htile = jnp.dot(hj_ref[...], r2_ref[...],
                    preferred_element_type=jnp.float32)
    prod = w2p * htile
    msg = jnp.dot(prod, s2_ref[...], preferred_element_type=jnp.float32)
    out_ref[...] = msg


def _tc_edge2(ea, hj, w1, b1, w2p, b2p, r2, s2):
    kk = 4000
    grid = _E // kk
    return pl.pallas_call(
        _edge2_body,
        grid=(grid,),
        in_specs=[
            pl.BlockSpec((kk, 1), lambda i: (i, 0)),
            pl.BlockSpec((kk, 16), lambda i: (i, 0)),
            pl.BlockSpec((1, 64), lambda i: (0, 0)),
            pl.BlockSpec((1, 64), lambda i: (0, 0)),
            pl.BlockSpec((64, 64), lambda i: (0, 0)),
            pl.BlockSpec((1, 64), lambda i: (0, 0)),
            pl.BlockSpec((16, 64), lambda i: (0, 0)),
            pl.BlockSpec((64, 16), lambda i: (0, 0)),
        ],
        out_specs=pl.BlockSpec((kk, 16), lambda i: (i, 0)),
        out_shape=jax.ShapeDtypeStruct((_E, 16), jnp.float32),
    )(ea, hj, w1, b1, w2p, b2p, r2, s2)


def _combine1_body(parts_ref, x_ref, rw_ref, b_ref, h1_ref, cnt_ref):
    sums = parts_ref[0] + parts_ref[1]
    cnt = jnp.maximum(sums[:, 8:9], 1.0)
    agg = sums[:, :8] / cnt
    root = jnp.dot(x_ref[...], rw_ref[...], preferred_element_type=jnp.float32)
    h1 = jnp.maximum(agg + root + b_ref[...], 0.0)
    h1_ref[...] = jnp.concatenate([h1, jnp.zeros((_N, 8), jnp.float32)],
                                  axis=1)
    cnt_ref[...] = cnt


def _tc_combine1(parts, x, rootw, bias):
    return pl.pallas_call(
        _combine1_body,
        out_shape=(jax.ShapeDtypeStruct((_N, 16), jnp.float32),
                   jax.ShapeDtypeStruct((_N, 1), jnp.float32)),
    )(parts, x, rootw, bias)


def _final_body(parts_ref, h1_ref, cnt_ref, batch_ref, rw_ref, b_ref,
                p1w_ref, p1b_ref, p2w_ref, p2b_ref, out_ref):
    sums = parts_ref[0][:, :8] + parts_ref[1][:, :8]
    agg = sums / cnt_ref[...]
    root = jnp.dot(h1_ref[:, :8], rw_ref[...],
                   preferred_element_type=jnp.float32)
    h2 = jnp.maximum(agg + root + b_ref[...], 0.0)
    gids = lax.broadcasted_iota(jnp.int32, (_N, 16), 1)
    oh = (batch_ref[...] == gids).astype(jnp.float32)
    hh = jnp.concatenate(
        [h2, jnp.ones((_N, 1), jnp.float32), jnp.zeros((_N, 7), jnp.float32)],
        axis=1)
    pooled_all = lax.dot_general(oh, hh, (((0,), (0,)), ((), ())),
                                 preferred_element_type=jnp.float32)
    cnt = jnp.maximum(pooled_all[:, 8:9], 1.0)
    pooled = pooled_all[:, :8] / cnt
    z = jnp.maximum(jnp.dot(pooled, p1w_ref[...],
                            preferred_element_type=jnp.float32)
                    + p1b_ref[...], 0.0)
    out_ref[...] = jnp.dot(z, p2w_ref[...],
                           preferred_element_type=jnp.float32) + p2b_ref[...]


def _tc_final(parts, h1pad, cnt, batch2d, rootw, bias, p1w, p1b, p2w, p2b):
    return pl.pallas_call(
        _final_body,
        out_shape=jax.ShapeDtypeStruct((16, 4), jnp.float32),
    )(parts, h1pad, cnt, batch2d, rootw, bias, p1w, p1b, p2w, p2b)


def kernel(x, edge_index, edge_attr, batch,
           nn1_W1, nn1_b1, nn1_W2, nn1_b2, conv1_root_W, conv1_bias,
           nn2_W1, nn2_b1, nn2_W2, nn2_b2, conv2_root_W, conv2_bias,
           proj1_W, proj1_b, proj2_W, proj2_b):
    src = edge_index[0]
    dst = edge_index[1]
    ea = edge_attr.reshape(_E, 1)

    a1 = nn1_W2.reshape(64, _DIN, 8).transpose(1, 2, 0).reshape(_DIN, 512)
    w1rep = jnp.tile(nn1_W1[0], 8).reshape(1, 512)
    b1rep = jnp.tile(nn1_b1, 8).reshape(1, 512)
    b2r = jnp.pad(nn1_b2.reshape(_DIN, 8), ((0, 0), (0, 8)))
    oid = jnp.arange(512) // 64
    s1 = (oid[:, None] == jnp.arange(16)[None, :]).astype(jnp.float32)
    w22p = nn2_W2.reshape(64, 8, 8).transpose(0, 2, 1).reshape(64, 64)
    b22p = nn2_b2.reshape(8, 8).transpose().reshape(1, 64)
    iid = jnp.arange(64) % 8
    r2 = ((jnp.arange(16)[:, None] == iid[None, :])
          & (jnp.arange(16)[:, None] < 8)).astype(jnp.float32)
    oid2 = jnp.arange(64) // 8
    s2 = (oid2[:, None] == jnp.arange(16)[None, :]).astype(jnp.float32)
    zrows = jnp.zeros((_ZROWS, 16), jnp.float32)
    batch2d = batch.reshape(_N, 1)

    xj = _sc_gather(x.astype(jnp.bfloat16), src, _DIN)
    msg1 = _tc_edge1(ea, xj, w1rep, b1rep, a1.astype(jnp.bfloat16),
                     b2r.astype(jnp.bfloat16), s1)
    parts1 = _sc_scatter(msg1, dst, zrows)
    h1pad, cnt = _tc_combine1(parts1, x, conv1_root_W,
                              conv1_bias.reshape(1, 8))
    h1j = _sc_gather(h1pad, src, 16)
    msg2 = _tc_edge2(ea, h1j, nn2_W1, nn2_b1.reshape(1, 64), w22p, b22p,
                     r2, s2)
    parts2 = _sc_scatter(msg2, dst, zrows)
    return _tc_final(parts2, h1pad, cnt, batch2d, conv2_root_W,
                     conv2_bias.reshape(1, 8), proj1_W,
                     proj1_b.reshape(1, 128), proj2_W, proj2_b.reshape(1, 4))


# speedup vs baseline: 1.2625x; 1.2625x over previous
import functools

import jax
import jax.numpy as jnp
from jax import lax
from jax.experimental import pallas as pl
from jax.experimental.pallas import tpu as pltpu
from jax.experimental.pallas import tpu_sc as plsc

_N = 10000
_E = 160000
_DIN = 128
_NW = 32
_PER_W = _E // _NW
_CH = 128
_NFULL = _PER_W // _CH
_TAIL = _PER_W - _NFULL * _CH
_ZROWS = 625


def _sc_gather(table, idx, d):
    dt = table.dtype
    mesh = plsc.VectorSubcoreMesh(core_axis_name="c", subcore_axis_name="s")

    @functools.partial(
        pl.kernel,
        out_type=jax.ShapeDtypeStruct((_E, d), dt),
        mesh=mesh,
        compiler_params=pltpu.CompilerParams(use_tc_tiling_on_sc=False),
        scratch_types=[
            pltpu.VMEM((_CH,), jnp.int32),
            pltpu.VMEM((_CH, d), dt),
            pltpu.VMEM((_TAIL,), jnp.int32),
            pltpu.VMEM((_TAIL, d), dt),
            pltpu.SemaphoreType.DMA,
        ],
    )
    def k(table_h, idx_h, out_h, idx_v, rows_v, idx_t, rows_t, sem):
        wid = lax.axis_index("s") * 2 + lax.axis_index("c")
        base = wid * _PER_W

        def body(j, carry):
            off = base + j * _CH
            pltpu.sync_copy(idx_h.at[pl.ds(off, _CH)], idx_v)
            pltpu.async_copy(table_h.at[idx_v], rows_v, sem).wait()
            pltpu.sync_copy(rows_v, out_h.at[pl.ds(off, _CH)])
            return carry

        lax.fori_loop(0, _NFULL, body, 0)
        off = base + _NFULL * _CH
        pltpu.sync_copy(idx_h.at[pl.ds(off, _TAIL)], idx_t)
        pltpu.async_copy(table_h.at[idx_t], rows_t, sem).wait()
        pltpu.sync_copy(rows_t, out_h.at[pl.ds(off, _TAIL)])

    return k(table, idx)


def _sc_scatter(msg, dst, zrows):
    mesh = plsc.VectorSubcoreMesh(core_axis_name="c", subcore_axis_name="s")

    @functools.partial(
        pl.kernel,
        out_type=jax.ShapeDtypeStruct((2, _N, 16), jnp.float32),
        mesh=mesh,
        compiler_params=pltpu.CompilerParams(use_tc_tiling_on_sc=False),
        scratch_types=[
            pltpu.VMEM((_CH,), jnp.int32),
            pltpu.VMEM((_CH, 16), jnp.float32),
            pltpu.VMEM((_TAIL,), jnp.int32),
            pltpu.VMEM((_TAIL, 16), jnp.float32),
            pltpu.VMEM_SHARED((_N, 16), jnp.float32),
        ],
    )
    def k(msg_h, dst_h, z_h, out_h, idx_v, msg_v, idx_t, msg_t, agg):
        cid = lax.axis_index("c")
        sid = lax.axis_index("s")
        pltpu.sync_copy(z_h, agg.at[pl.ds(sid * _ZROWS, _ZROWS)])
        plsc.subcore_barrier()

        wid = sid * 2 + cid
        base = wid * _PER_W

        def body(j, carry):
            off = base + j * _CH
            pltpu.sync_copy(dst_h.at[pl.ds(off, _CH)], idx_v)
            pltpu.sync_copy(msg_h.at[pl.ds(off, _CH)], msg_v)
            pltpu.sync_copy(msg_v, agg.at[idx_v], add=True)
            return carry

        lax.fori_loop(0, _NFULL, body, 0)
        off = base + _NFULL * _CH
        pltpu.sync_copy(dst_h.at[pl.ds(off, _TAIL)], idx_t)
        pltpu.sync_copy(msg_h.at[pl.ds(off, _TAIL)], msg_t)
        pltpu.sync_copy(msg_t, agg.at[idx_t], add=True)

        plsc.subcore_barrier()

        @pl.when(sid == 0)
        def _():
            pltpu.sync_copy(agg, out_h.at[cid])

    return k(msg, dst, zrows)


def _edge1_body(ea_ref, xj_ref, w1r_ref, b1r_ref, a1_ref, b2r_ref, s1_ref,
                out_ref):
    ea = ea_ref[...]
    hrep = jnp.maximum(ea * w1r_ref[...] + b1r_ref[...], 0.0)
    g = jnp.dot(xj_ref[...], a1_ref[...],
                preferred_element_type=jnp.float32)
    prod = hrep * g
    msg = jnp.dot(prod, s1_ref[...], preferred_element_type=jnp.float32)
    msg = msg + jnp.dot(xj_ref[...], b2r_ref[...],
                        preferred_element_type=jnp.float32)
    k = msg.shape[0]
    col = lax.broadcasted_iota(jnp.int32, (k, 16), 1)
    out_ref[...] = msg + (col == 8).astype(jnp.float32)


def _tc_edge1(ea, xj, w1rep, b1rep, a1, b2r, s1):
    kk = 4000
    grid = _E // kk
    return pl.pallas_call(
        _edge1_body,
        grid=(grid,),
        in_specs=[
            pl.BlockSpec((kk, 1), lambda i: (i, 0)),
            pl.BlockSpec((kk, _DIN), lambda i: (i, 0)),
            pl.BlockSpec((1, 512), lambda i: (0, 0)),
            pl.BlockSpec((1, 512), lambda i: (0, 0)),
            pl.BlockSpec((_DIN, 512), lambda i: (0, 0)),
            pl.BlockSpec((_DIN, 16), lambda i: (0, 0)),
            pl.BlockSpec((512, 16), lambda i: (0, 0)),
        ],
        out_specs=pl.BlockSpec((kk, 16), lambda i: (i, 0)),
        out_shape=jax.ShapeDtypeStruct((_E, 16), jnp.float32),
    )(ea, xj, w1rep, b1rep, a1, b2r, s1)


def _edge2_body(ea_ref, hj_ref, w1_ref, b1_ref, w2p_ref, b2p_ref, r2_ref,
                s2_ref, out_ref):
    ea = ea_ref[...]
    h2 = jnp.maximum(ea * w1_ref[...] + b1_ref[...], 0.0)
    w2p = jnp.dot(h2, w2p_ref[...],
                  preferred_element_type=jnp.float32) + b2p_ref[...]
    htile = jnp.dot(hj_ref[...], r2_ref[...],
                    preferred_element_type=jnp.float32)
    prod = w2p * htile
    msg = jnp.dot(prod, s2_ref[...], preferred_element_type=jnp.float32)
    out_ref[...] = msg


def _tc_edge2(ea, hj, w1, b1, w2p, b2p, r2, s2):
    kk = 8000
    grid = _E // kk
    return pl.pallas_call(
        _edge2_body,
        grid=(grid,),
        in_specs=[
            pl.BlockSpec((kk, 1), lambda i: (i, 0)),
            pl.BlockSpec((kk, 16), lambda i: (i, 0)),
            pl.BlockSpec((1, 64), lambda i: (0, 0)),
            pl.BlockSpec((1, 64), lambda i: (0, 0)),
            pl.BlockSpec((64, 64), lambda i: (0, 0)),
            pl.BlockSpec((1, 64), lambda i: (0, 0)),
            pl.BlockSpec((16, 64), lambda i: (0, 0)),
            pl.BlockSpec((64, 16), lambda i: (0, 0)),
        ],
        out_specs=pl.BlockSpec((kk, 16), lambda i: (i, 0)),
        out_shape=jax.ShapeDtypeStruct((_E, 16), jnp.float32),
    )(ea, hj, w1, b1, w2p, b2p, r2, s2)


def _combine1_body(parts_ref, x_ref, rw_ref, b_ref, h1_ref, cnt_ref):
    sums = parts_ref[0] + parts_ref[1]
    cnt = jnp.maximum(sums[:, 8:9], 1.0)
    agg = sums[:, :8] / cnt
    root = jnp.dot(x_ref[...], rw_ref[...], preferred_element_type=jnp.float32)
    h1 = jnp.maximum(agg + root + b_ref[...], 0.0)
    h1_ref[...] = jnp.concatenate([h1, jnp.zeros((_N, 8), jnp.float32)],
                                  axis=1)
    cnt_ref[...] = cnt


def _tc_combine1(parts, x, rootw, bias):
    return pl.pallas_call(
        _combine1_body,
        out_shape=(jax.ShapeDtypeStruct((_N, 16), jnp.float32),
                   jax.ShapeDtypeStruct((_N, 1), jnp.float32)),
    )(parts, x, rootw, bias)


def _final_body(parts_ref, h1_ref, cnt_ref, batch_ref, rw_ref, b_ref,
                p1w_ref, p1b_ref, p2w_ref, p2b_ref, out_ref):
    sums = parts_ref[0][:, :8] + parts_ref[1][:, :8]
    agg = sums / cnt_ref[...]
    root = jnp.dot(h1_ref[:, :8], rw_ref[...],
                   preferred_element_type=jnp.float32)
    h2 = jnp.maximum(agg + root + b_ref[...], 0.0)
    gids = lax.broadcasted_iota(jnp.int32, (_N, 16), 1)
    oh = (batch_ref[...] == gids).astype(jnp.float32)
    hh = jnp.concatenate(
        [h2, jnp.ones((_N, 1), jnp.float32), jnp.zeros((_N, 7), jnp.float32)],
        axis=1)
    pooled_all = lax.dot_general(oh, hh, (((0,), (0,)), ((), ())),
                                 preferred_element_type=jnp.float32)
    cnt = jnp.maximum(pooled_all[:, 8:9], 1.0)
    pooled = pooled_all[:, :8] / cnt
    z = jnp.maximum(jnp.dot(pooled, p1w_ref[...],
                            preferred_element_type=jnp.float32)
                    + p1b_ref[...], 0.0)
    out_ref[...] = jnp.dot(z, p2w_ref[...],
                           preferred_element_type=jnp.float32) + p2b_ref[...]


def _tc_final(parts, h1pad, cnt, batch2d, rootw, bias, p1w, p1b, p2w, p2b):
    return pl.pallas_call(
        _final_body,
        out_shape=jax.ShapeDtypeStruct((16, 4), jnp.float32),
    )(parts, h1pad, cnt, batch2d, rootw, bias, p1w, p1b, p2w, p2b)


def kernel(x, edge_index, edge_attr, batch,
           nn1_W1, nn1_b1, nn1_W2, nn1_b2, conv1_root_W, conv1_bias,
           nn2_W1, nn2_b1, nn2_W2, nn2_b2, conv2_root_W, conv2_bias,
           proj1_W, proj1_b, proj2_W, proj2_b):
    src = edge_index[0]
    dst = edge_index[1]
    ea = edge_attr.reshape(_E, 1)

    a1 = nn1_W2.reshape(64, _DIN, 8).transpose(1, 2, 0).reshape(_DIN, 512)
    w1rep = jnp.tile(nn1_W1[0], 8).reshape(1, 512)
    b1rep = jnp.tile(nn1_b1, 8).reshape(1, 512)
    b2r = jnp.pad(nn1_b2.reshape(_DIN, 8), ((0, 0), (0, 8)))
    oid = jnp.arange(512) // 64
    s1 = (oid[:, None] == jnp.arange(16)[None, :]).astype(jnp.float32)
    w22p = nn2_W2.reshape(64, 8, 8).transpose(0, 2, 1).reshape(64, 64)
    b22p = nn2_b2.reshape(8, 8).transpose().reshape(1, 64)
    iid = jnp.arange(64) % 8
    r2 = ((jnp.arange(16)[:, None] == iid[None, :])
          & (jnp.arange(16)[:, None] < 8)).astype(jnp.float32)
    oid2 = jnp.arange(64) // 8
    s2 = (oid2[:, None] == jnp.arange(16)[None, :]).astype(jnp.float32)
    zrows = jnp.zeros((_ZROWS, 16), jnp.float32)
    batch2d = batch.reshape(_N, 1)

    xj = _sc_gather(x, src, _DIN)
    msg1 = _tc_edge1(ea, xj, w1rep, b1rep, a1, b2r, s1)
    parts1 = _sc_scatter(msg1, dst, zrows)
    h1pad, cnt = _tc_combine1(parts1, x, conv1_root_W,
                              conv1_bias.reshape(1, 8))
    h1j = _sc_gather(h1pad, src, 16)
    msg2 = _tc_edge2(ea, h1j, nn2_W1, nn2_b1.reshape(1, 64), w22p, b22p,
                     r2, s2)
    parts2 = _sc_scatter(msg2, dst, zrows)
    return _tc_final(parts2, h1pad, cnt, batch2d, conv2_root_W,
                     conv2_bias.reshape(1, 8), proj1_W,
                     proj1_b.reshape(1, 128), proj2_W, proj2_b.reshape(1, 4))


# speedup vs baseline: 1.4878x; 1.1785x over previous
import functools

import jax
import jax.numpy as jnp
from jax import lax
from jax.experimental import pallas as pl
from jax.experimental.pallas import tpu as pltpu
from jax.experimental.pallas import tpu_sc as plsc

_N = 10000
_E = 160000
_DIN = 128
_NW = 32
_PER_W = _E // _NW
_CH = 112
_NFULL = 44
_TAIL = _PER_W - _NFULL * _CH
_ZROWS = 625


def _sc_gather(table, idx, d):
    dt = table.dtype
    mesh = plsc.VectorSubcoreMesh(core_axis_name="c", subcore_axis_name="s")

    @functools.partial(
        pl.kernel,
        out_type=jax.ShapeDtypeStruct((_E, d), dt),
        mesh=mesh,
        compiler_params=pltpu.CompilerParams(use_tc_tiling_on_sc=False),
        scratch_types=[
            [pltpu.VMEM((_CH,), jnp.int32)] * 2,
            [pltpu.VMEM((_CH, d), dt)] * 2,
            pltpu.VMEM((_TAIL,), jnp.int32),
            pltpu.VMEM((_TAIL, d), dt),
            [pltpu.SemaphoreType.DMA] * 2,
            [pltpu.SemaphoreType.DMA] * 2,
            [pltpu.SemaphoreType.DMA] * 2,
        ],
    )
    def k(table_h, idx_h, out_h, idx_b, rows_b, idx_t, rows_t,
          sem_i, sem_g, sem_w):
        wid = lax.axis_index("s") * 2 + lax.axis_index("c")
        base = wid * _PER_W

        def start_idx(j, b):
            pltpu.async_copy(idx_h.at[pl.ds(base + j * _CH, _CH)],
                             idx_b[b], sem_i[b])

        def wait_idx(j, b):
            pltpu.make_async_copy(idx_h.at[pl.ds(base + j * _CH, _CH)],
                                  idx_b[b], sem_i[b]).wait()

        def start_write(j, b):
            pltpu.async_copy(rows_b[b], out_h.at[pl.ds(base + j * _CH, _CH)],
                             sem_w[b])

        def wait_write(j, b):
            pltpu.make_async_copy(rows_b[b],
                                  out_h.at[pl.ds(base + j * _CH, _CH)],
                                  sem_w[b]).wait()

        start_idx(0, 0)
        start_idx(1, 1)

        def body(p, carry):
            for b in range(2):
                j = 2 * p + b
                wait_idx(j, b)

                @pl.when(p > 0)
                def _():
                    wait_write(j - 2, b)

                pltpu.async_copy(table_h.at[idx_b[b]], rows_b[b],
                                 sem_g[b]).wait()
                start_write(j, b)

                @pl.when(p < _NFULL // 2 - 1)
                def _():
                    start_idx(j + 2, b)
            return carry

        lax.fori_loop(0, _NFULL // 2, body, 0)
        wait_write(_NFULL - 2, 0)
        wait_write(_NFULL - 1, 1)
        off = base + _NFULL * _CH
        pltpu.sync_copy(idx_h.at[pl.ds(off, _TAIL)], idx_t)
        pltpu.async_copy(table_h.at[idx_t], rows_t, sem_g[0]).wait()
        pltpu.sync_copy(rows_t, out_h.at[pl.ds(off, _TAIL)])

    return k(table, idx)


def _sc_scatter(msg, dst, zrows):
    mesh = plsc.VectorSubcoreMesh(core_axis_name="c", subcore_axis_name="s")

    @functools.partial(
        pl.kernel,
        out_type=jax.ShapeDtypeStruct((2, _N, 16), jnp.float32),
        mesh=mesh,
        compiler_params=pltpu.CompilerParams(use_tc_tiling_on_sc=False),
        scratch_types=[
            [pltpu.VMEM((_CH,), jnp.int32)] * 2,
            [pltpu.VMEM((_CH, 16), jnp.float32)] * 2,
            pltpu.VMEM((_TAIL,), jnp.int32),
            pltpu.VMEM((_TAIL, 16), jnp.float32),
            pltpu.VMEM_SHARED((_N, 16), jnp.float32),
            [pltpu.SemaphoreType.DMA] * 2,
            [pltpu.SemaphoreType.DMA] * 2,
        ],
    )
    def k(msg_h, dst_h, z_h, out_h, idx_b, msg_b, idx_t, msg_t, agg,
          sem_i, sem_m):
        cid = lax.axis_index("c")
        sid = lax.axis_index("s")
        pltpu.sync_copy(z_h, agg.at[pl.ds(sid * _ZROWS, _ZROWS)])
        plsc.subcore_barrier()

        wid = sid * 2 + cid
        base = wid * _PER_W

        def start_load(j, b):
            off = base + j * _CH
            pltpu.async_copy(dst_h.at[pl.ds(off, _CH)], idx_b[b], sem_i[b])
            pltpu.async_copy(msg_h.at[pl.ds(off, _CH)], msg_b[b], sem_m[b])

        def wait_load(j, b):
            off = base + j * _CH
            pltpu.make_async_copy(dst_h.at[pl.ds(off, _CH)], idx_b[b],
                                  sem_i[b]).wait()
            pltpu.make_async_copy(msg_h.at[pl.ds(off, _CH)], msg_b[b],
                                  sem_m[b]).wait()

        start_load(0, 0)
        start_load(1, 1)

        def body(p, carry):
            for b in range(2):
                j = 2 * p + b
                wait_load(j, b)
                pltpu.sync_copy(msg_b[b], agg.at[idx_b[b]], add=True)

                @pl.when(p < _NFULL // 2 - 1)
                def _():
                    start_load(j + 2, b)
            return carry

        lax.fori_loop(0, _NFULL // 2, body, 0)
        off = base + _NFULL * _CH
        pltpu.sync_copy(dst_h.at[pl.ds(off, _TAIL)], idx_t)
        pltpu.sync_copy(msg_h.at[pl.ds(off, _TAIL)], msg_t)
        pltpu.sync_copy(msg_t, agg.at[idx_t], add=True)

        plsc.subcore_barrier()

        @pl.when(sid == 0)
        def _():
            pltpu.sync_copy(agg, out_h.at[cid])

    return k(msg, dst, zrows)


def _edge1_body(ea_ref, xj_ref, w1r_ref, b1r_ref, a1_ref, b2r_ref, s1_ref,
                out_ref):
    ea = ea_ref[...]
    hrep = jnp.maximum(ea * w1r_ref[...] + b1r_ref[...], 0.0)
    g = jnp.dot(xj_ref[...], a1_ref[...],
                preferred_element_type=jnp.float32)
    prod = hrep * g
    msg = jnp.dot(prod, s1_ref[...], preferred_element_type=jnp.float32)
    msg = msg + jnp.dot(xj_ref[...], b2r_ref[...],
                        preferred_element_type=jnp.float32)
    k = msg.shape[0]
    col = lax.broadcasted_iota(jnp.int32, (k, 16), 1)
    out_ref[...] = msg + (col == 8).astype(jnp.float32)


def _tc_edge1(ea, xj, w1rep, b1rep, a1, b2r, s1):
    kk = 4000
    grid = _E // kk
    return pl.pallas_call(
        _edge1_body,
        grid=(grid,),
        in_specs=[
            pl.BlockSpec((kk, 1), lambda i: (i, 0)),
            pl.BlockSpec((kk, _DIN), lambda i: (i, 0)),
            pl.BlockSpec((1, 512), lambda i: (0, 0)),
            pl.BlockSpec((1, 512), lambda i: (0, 0)),
            pl.BlockSpec((_DIN, 512), lambda i: (0, 0)),
            pl.BlockSpec((_DIN, 16), lambda i: (0, 0)),
            pl.BlockSpec((512, 16), lambda i: (0, 0)),
        ],
        out_specs=pl.BlockSpec((kk, 16), lambda i: (i, 0)),
        out_shape=jax.ShapeDtypeStruct((_E, 16), jnp.float32),
    )(ea, xj, w1rep, b1rep, a1, b2r, s1)


def _edge2_body(ea_ref, hj_ref, w1_ref, b1_ref, w2p_ref, b2p_ref, r2_ref,
                s2_ref, out_ref):
    ea = ea_ref[...]
    h2 = jnp.maximum(ea * w1_ref[...] + b1_ref[...], 0.0)
    w2p = jnp.dot(h2, w2p_ref[...],
                  preferred_element_type=jnp.float32) + b2p_ref[...]
    htile = jnp.dot(hj_ref[...], r2_ref[...],
                    preferred_element_type=jnp.float32)
    prod = w2p * htile
    msg = jnp.dot(prod, s2_ref[...], preferred_element_type=jnp.float32)
    out_ref[...] = msg


def _tc_edge2(ea, hj, w1, b1, w2p, b2p, r2, s2):
    kk = 8000
    grid = _E // kk
    return pl.pallas_call(
        _edge2_body,
        grid=(grid,),
        in_specs=[
            pl.BlockSpec((kk, 1), lambda i: (i, 0)),
            pl.BlockSpec((kk, 16), lambda i: (i, 0)),
            pl.BlockSpec((1, 64), lambda i: (0, 0)),
            pl.BlockSpec((1, 64), lambda i: (0, 0)),
            pl.BlockSpec((64, 64), lambda i: (0, 0)),
            pl.BlockSpec((1, 64), lambda i: (0, 0)),
            pl.BlockSpec((16, 64), lambda i: (0, 0)),
            pl.BlockSpec((64, 16), lambda i: (0, 0)),
        ],
        out_specs=pl.BlockSpec((kk, 16), lambda i: (i, 0)),
        out_shape=jax.ShapeDtypeStruct((_E, 16), jnp.float32),
    )(ea, hj, w1, b1, w2p, b2p, r2, s2)


def _combine1_body(parts_ref, x_ref, rw_ref, b_ref, h1_ref, cnt_ref):
    sums = parts_ref[0] + parts_ref[1]
    cnt = jnp.maximum(sums[:, 8:9], 1.0)
    agg = sums[:, :8] / cnt
    root = jnp.dot(x_ref[...], rw_ref[...], preferred_element_type=jnp.float32)
    h1 = jnp.maximum(agg + root + b_ref[...], 0.0)
    h1_ref[...] = jnp.concatenate([h1, jnp.zeros((_N, 8), jnp.float32)],
                                  axis=1)
    cnt_ref[...] = cnt


def _tc_combine1(parts, x, rootw, bias):
    return pl.pallas_call(
        _combine1_body,
        out_shape=(jax.ShapeDtypeStruct((_N, 16), jnp.float32),
                   jax.ShapeDtypeStruct((_N, 1), jnp.float32)),
    )(parts, x, rootw, bias)


def _final_body(parts_ref, h1_ref, cnt_ref, batch_ref, rw_ref, b_ref,
                p1w_ref, p1b_ref, p2w_ref, p2b_ref, out_ref):
    sums = parts_ref[0][:, :8] + parts_ref[1][:, :8]
    agg = sums / cnt_ref[...]
    root = jnp.dot(h1_ref[:, :8], rw_ref[...],
                   preferred_element_type=jnp.float32)
    h2 = jnp.maximum(agg + root + b_ref[...], 0.0)
    gids = lax.broadcasted_iota(jnp.int32, (_N, 16), 1)
    oh = (batch_ref[...] == gids).astype(jnp.float32)
    hh = jnp.concatenate(
        [h2, jnp.ones((_N, 1), jnp.float32), jnp.zeros((_N, 7), jnp.float32)],
        axis=1)
    pooled_all = lax.dot_general(oh, hh, (((0,), (0,)), ((), ())),
                                 preferred_element_type=jnp.float32)
    cnt = jnp.maximum(pooled_all[:, 8:9], 1.0)
    pooled = pooled_all[:, :8] / cnt
    z = jnp.maximum(jnp.dot(pooled, p1w_ref[...],
                            preferred_element_type=jnp.float32)
                    + p1b_ref[...], 0.0)
    out_ref[...] = jnp.dot(z, p2w_ref[...],
                           preferred_element_type=jnp.float32) + p2b_ref[...]


def _tc_final(parts, h1pad, cnt, batch2d, rootw, bias, p1w, p1b, p2w, p2b):
    return pl.pallas_call(
        _final_body,
        out_shape=jax.ShapeDtypeStruct((16, 4), jnp.float32),
    )(parts, h1pad, cnt, batch2d, rootw, bias, p1w, p1b, p2w, p2b)


def kernel(x, edge_index, edge_attr, batch,
           nn1_W1, nn1_b1, nn1_W2, nn1_b2, conv1_root_W, conv1_bias,
           nn2_W1, nn2_b1, nn2_W2, nn2_b2, conv2_root_W, conv2_bias,
           proj1_W, proj1_b, proj2_W, proj2_b):
    src = edge_index[0]
    dst = edge_index[1]
    ea = edge_attr.reshape(_E, 1)

    a1 = nn1_W2.reshape(64, _DIN, 8).transpose(1, 2, 0).reshape(_DIN, 512)
    w1rep = jnp.tile(nn1_W1[0], 8).reshape(1, 512)
    b1rep = jnp.tile(nn1_b1, 8).reshape(1, 512)
    b2r = jnp.pad(nn1_b2.reshape(_DIN, 8), ((0, 0), (0, 8)))
    oid = jnp.arange(512) // 64
    s1 = (oid[:, None] == jnp.arange(16)[None, :]).astype(jnp.float32)
    w22p = nn2_W2.reshape(64, 8, 8).transpose(0, 2, 1).reshape(64, 64)
    b22p = nn2_b2.reshape(8, 8).transpose().reshape(1, 64)
    iid = jnp.arange(64) % 8
    r2 = ((jnp.arange(16)[:, None] == iid[None, :])
          & (jnp.arange(16)[:, None] < 8)).astype(jnp.float32)
    oid2 = jnp.arange(64) // 8
    s2 = (oid2[:, None] == jnp.arange(16)[None, :]).astype(jnp.float32)
    zrows = jnp.zeros((_ZROWS, 16), jnp.float32)
    batch2d = batch.reshape(_N, 1)

    xj = _sc_gather(x, src, _DIN)
    msg1 = _tc_edge1(ea, xj, w1rep, b1rep, a1, b2r, s1)
    parts1 = _sc_scatter(msg1, dst, zrows)
    h1pad, cnt = _tc_combine1(parts1, x, conv1_root_W,
                              conv1_bias.reshape(1, 8))
    h1j = _sc_gather(h1pad, src, 16)
    msg2 = _tc_edge2(ea, h1j, nn2_W1, nn2_b1.reshape(1, 64), w22p, b22p,
                     r2, s2)
    parts2 = _sc_scatter(msg2, dst, zrows)
    return _tc_final(parts2, h1pad, cnt, batch2d, conv2_root_W,
                     conv2_bias.reshape(1, 8), proj1_W,
                     proj1_b.reshape(1, 128), proj2_W, proj2_b.reshape(1, 4))


# speedup vs baseline: 1.5017x; 1.0093x over previous
import functools

import jax
import jax.numpy as jnp
from jax import lax
from jax.experimental import pallas as pl
from jax.experimental.pallas import tpu as pltpu
from jax.experimental.pallas import tpu_sc as plsc

_N = 10000
_E = 160000
_DIN = 128
_NW = 32
_PER_W = _E // _NW
_CH = 112
_NFULL = 44
_TAIL = _PER_W - _NFULL * _CH
_ZROWS = 625


def _sc_gather(table, idx, d):
    dt = table.dtype
    mesh = plsc.VectorSubcoreMesh(core_axis_name="c", subcore_axis_name="s")

    @functools.partial(
        pl.kernel,
        out_type=jax.ShapeDtypeStruct((_E, d), dt),
        mesh=mesh,
        compiler_params=pltpu.CompilerParams(use_tc_tiling_on_sc=False),
        scratch_types=[
            [pltpu.VMEM((_CH,), jnp.int32)] * 2,
            [pltpu.VMEM((_CH, d), dt)] * 2,
            pltpu.VMEM((_TAIL,), jnp.int32),
            pltpu.VMEM((_TAIL, d), dt),
            [pltpu.SemaphoreType.DMA] * 2,
            [pltpu.SemaphoreType.DMA] * 2,
            [pltpu.SemaphoreType.DMA] * 2,
        ],
    )
    def k(table_h, idx_h, out_h, idx_b, rows_b, idx_t, rows_t,
          sem_i, sem_g, sem_w):
        wid = lax.axis_index("s") * 2 + lax.axis_index("c")
        base = wid * _PER_W

        def start_idx(j, b):
            pltpu.async_copy(idx_h.at[pl.ds(base + j * _CH, _CH)],
                             idx_b[b], sem_i[b])

        def wait_idx(j, b):
            pltpu.make_async_copy(idx_h.at[pl.ds(base + j * _CH, _CH)],
                                  idx_b[b], sem_i[b]).wait()

        def start_write(j, b):
            pltpu.async_copy(rows_b[b], out_h.at[pl.ds(base + j * _CH, _CH)],
                             sem_w[b])

        def wait_write(j, b):
            pltpu.make_async_copy(rows_b[b],
                                  out_h.at[pl.ds(base + j * _CH, _CH)],
                                  sem_w[b]).wait()

        start_idx(0, 0)
        start_idx(1, 1)

        def body(p, carry):
            for b in range(2):
                j = 2 * p + b
                wait_idx(j, b)

                @pl.when(p > 0)
                def _():
                    wait_write(j - 2, b)

                pltpu.async_copy(table_h.at[idx_b[b]], rows_b[b],
                                 sem_g[b]).wait()
                start_write(j, b)

                @pl.when(p < _NFULL // 2 - 1)
                def _():
                    start_idx(j + 2, b)
            return carry

        lax.fori_loop(0, _NFULL // 2, body, 0)
        wait_write(_NFULL - 2, 0)
        wait_write(_NFULL - 1, 1)
        off = base + _NFULL * _CH
        pltpu.sync_copy(idx_h.at[pl.ds(off, _TAIL)], idx_t)
        pltpu.async_copy(table_h.at[idx_t], rows_t, sem_g[0]).wait()
        pltpu.sync_copy(rows_t, out_h.at[pl.ds(off, _TAIL)])

    return k(table, idx)


def _sc_scatter(msg, dst, zrows):
    mesh = plsc.VectorSubcoreMesh(core_axis_name="c", subcore_axis_name="s")

    @functools.partial(
        pl.kernel,
        out_type=jax.ShapeDtypeStruct((2, _N, 16), jnp.float32),
        mesh=mesh,
        compiler_params=pltpu.CompilerParams(use_tc_tiling_on_sc=False),
        scratch_types=[
            [pltpu.VMEM((_CH,), jnp.int32)] * 2,
            [pltpu.VMEM((_CH, 16), jnp.float32)] * 2,
            pltpu.VMEM((_TAIL,), jnp.int32),
            pltpu.VMEM((_TAIL, 16), jnp.float32),
            pltpu.VMEM_SHARED((_N, 16), jnp.float32),
            [pltpu.SemaphoreType.DMA] * 2,
            [pltpu.SemaphoreType.DMA] * 2,
        ],
    )
    def k(msg_h, dst_h, z_h, out_h, idx_b, msg_b, idx_t, msg_t, agg,
          sem_i, sem_m):
        cid = lax.axis_index("c")
        sid = lax.axis_index("s")
        pltpu.sync_copy(z_h, agg.at[pl.ds(sid * _ZROWS, _ZROWS)])
        plsc.subcore_barrier()

        wid = sid * 2 + cid
        base = wid * _PER_W

        def start_load(j, b):
            off = base + j * _CH
            pltpu.async_copy(dst_h.at[pl.ds(off, _CH)], idx_b[b], sem_i[b])
            pltpu.async_copy(msg_h.at[pl.ds(off, _CH)], msg_b[b], sem_m[b])

        def wait_load(j, b):
            off = base + j * _CH
            pltpu.make_async_copy(dst_h.at[pl.ds(off, _CH)], idx_b[b],
                                  sem_i[b]).wait()
            pltpu.make_async_copy(msg_h.at[pl.ds(off, _CH)], msg_b[b],
                                  sem_m[b]).wait()

        start_load(0, 0)
        start_load(1, 1)

        def body(p, carry):
            for b in range(2):
                j = 2 * p + b
                wait_load(j, b)
                pltpu.sync_copy(msg_b[b], agg.at[idx_b[b]], add=True)

                @pl.when(p < _NFULL // 2 - 1)
                def _():
                    start_load(j + 2, b)
            return carry

        lax.fori_loop(0, _NFULL // 2, body, 0)
        off = base + _NFULL * _CH
        pltpu.sync_copy(dst_h.at[pl.ds(off, _TAIL)], idx_t)
        pltpu.sync_copy(msg_h.at[pl.ds(off, _TAIL)], msg_t)
        pltpu.sync_copy(msg_t, agg.at[idx_t], add=True)

        plsc.subcore_barrier()

        @pl.when(sid == 0)
        def _():
            pltpu.sync_copy(agg, out_h.at[cid])

    return k(msg, dst, zrows)


def _edge1_body(ea_ref, xj_ref, w1r_ref, b1r_ref, a1_ref, b2r_ref, s1_ref,
                out_ref):
    ea = ea_ref[...]
    hrep = jnp.maximum(ea * w1r_ref[...] + b1r_ref[...], 0.0)
    g = jnp.dot(xj_ref[...], a1_ref[...],
                preferred_element_type=jnp.float32)
    prod = hrep * g
    msg = jnp.dot(prod, s1_ref[...], preferred_element_type=jnp.float32)
    msg = msg + jnp.dot(xj_ref[...], b2r_ref[...],
                        preferred_element_type=jnp.float32)
    k = msg.shape[0]
    col = lax.broadcasted_iota(jnp.int32, (k, 16), 1)
    out_ref[...] = msg + (col == 8).astype(jnp.float32)


def _tc_edge1(ea, xj, w1rep, b1rep, a1, b2r, s1):
    kk = 4000
    grid = _E // kk
    return pl.pallas_call(
        _edge1_body,
        grid=(grid,),
        in_specs=[
            pl.BlockSpec((kk, 1), lambda i: (i, 0)),
            pl.BlockSpec((kk, _DIN), lambda i: (i, 0)),
            pl.BlockSpec((1, 512), lambda i: (0, 0)),
            pl.BlockSpec((1, 512), lambda i: (0, 0)),
            pl.BlockSpec((_DIN, 512), lambda i: (0, 0)),
            pl.BlockSpec((_DIN, 16), lambda i: (0, 0)),
            pl.BlockSpec((512, 16), lambda i: (0, 0)),
        ],
        out_specs=pl.BlockSpec((kk, 16), lambda i: (i, 0)),
        out_shape=jax.ShapeDtypeStruct((_E, 16), jnp.float32),
    )(ea, xj, w1rep, b1rep, a1, b2r, s1)


def _edge2_body(ea_ref, hj_ref, w1_ref, b1_ref, w2p_ref, b2p_ref, r2_ref,
                s2_ref, out_ref):
    ea = ea_ref[...]
    h2 = jnp.maximum(ea * w1_ref[...] + b1_ref[...], 0.0)
    w2p = jnp.dot(h2, w2p_ref[...],
                  preferred_element_type=jnp.float32) + b2p_ref[...]
    htile = jnp.dot(hj_ref[...], r2_ref[...],
                    preferred_element_type=jnp.float32)
    prod = w2p * htile
    msg = jnp.dot(prod, s2_ref[...], preferred_element_type=jnp.float32)
    out_ref[...] = msg


def _tc_edge2(ea, hj, w1, b1, w2p, b2p, r2, s2):
    kk = 8000
    grid = _E // kk
    return pl.pallas_call(
        _edge2_body,
        grid=(grid,),
        in_specs=[
            pl.BlockSpec((kk, 1), lambda i: (i, 0)),
            pl.BlockSpec((kk, 16), lambda i: (i, 0)),
            pl.BlockSpec((1, 64), lambda i: (0, 0)),
            pl.BlockSpec((1, 64), lambda i: (0, 0)),
            pl.BlockSpec((64, 64), lambda i: (0, 0)),
            pl.BlockSpec((1, 64), lambda i: (0, 0)),
            pl.BlockSpec((16, 64), lambda i: (0, 0)),
            pl.BlockSpec((64, 16), lambda i: (0, 0)),
        ],
        out_specs=pl.BlockSpec((kk, 16), lambda i: (i, 0)),
        out_shape=jax.ShapeDtypeStruct((_E, 16), jnp.float32),
    )(ea, hj, w1, b1, w2p, b2p, r2, s2)


def _combine1_body(parts_ref, x_ref, rw_ref, b_ref, h1_ref, cnt_ref):
    sums = parts_ref[0] + parts_ref[1]
    cnt = jnp.maximum(sums[:, 8:9], 1.0)
    agg = sums[:, :8] / cnt
    root = jnp.dot(x_ref[...], rw_ref[...], preferred_element_type=jnp.float32)
    h1 = jnp.maximum(agg + root + b_ref[...], 0.0)
    h1_ref[...] = jnp.concatenate([h1, jnp.zeros((_N, 8), jnp.float32)],
                                  axis=1)
    cnt_ref[...] = cnt


def _tc_combine1(parts, x, rootw, bias):
    return pl.pallas_call(
        _combine1_body,
        out_shape=(jax.ShapeDtypeStruct((_N, 16), jnp.float32),
                   jax.ShapeDtypeStruct((_N, 1), jnp.float32)),
    )(parts, x, rootw, bias)


def _final_body(parts_ref, h1_ref, cnt_ref, batch_ref, rw_ref, b_ref,
                p1w_ref, p1b_ref, p2w_ref, p2b_ref, out_ref):
    sums = parts_ref[0][:, :8] + parts_ref[1][:, :8]
    agg = sums / cnt_ref[...]
    root = jnp.dot(h1_ref[:, :8], rw_ref[...],
                   preferred_element_type=jnp.float32)
    h2 = jnp.maximum(agg + root + b_ref[...], 0.0)
    gids = lax.broadcasted_iota(jnp.int32, (16, _N), 0)
    oht = (batch_ref[...] == gids).astype(jnp.float32)
    hh = jnp.concatenate(
        [h2, jnp.ones((_N, 1), jnp.float32), jnp.zeros((_N, 7), jnp.float32)],
        axis=1)
    pooled_all = jnp.dot(oht, hh, preferred_element_type=jnp.float32)
    cnt = jnp.maximum(pooled_all[:, 8:9], 1.0)
    pooled = pooled_all[:, :8] / cnt
    z = jnp.maximum(jnp.dot(pooled, p1w_ref[...],
                            preferred_element_type=jnp.float32)
                    + p1b_ref[...], 0.0)
    out_ref[...] = jnp.dot(z, p2w_ref[...],
                           preferred_element_type=jnp.float32) + p2b_ref[...]


def _tc_final(parts, h1pad, cnt, batch2d, rootw, bias, p1w, p1b, p2w, p2b):
    return pl.pallas_call(
        _final_body,
        out_shape=jax.ShapeDtypeStruct((16, 4), jnp.float32),
    )(parts, h1pad, cnt, batch2d, rootw, bias, p1w, p1b, p2w, p2b)


def kernel(x, edge_index, edge_attr, batch,
           nn1_W1, nn1_b1, nn1_W2, nn1_b2, conv1_root_W, conv1_bias,
           nn2_W1, nn2_b1, nn2_W2, nn2_b2, conv2_root_W, conv2_bias,
           proj1_W, proj1_b, proj2_W, proj2_b):
    src = edge_index[0]
    dst = edge_index[1]
    ea = edge_attr.reshape(_E, 1)

    a1 = nn1_W2.reshape(64, _DIN, 8).transpose(1, 2, 0).reshape(_DIN, 512)
    w1rep = jnp.tile(nn1_W1[0], 8).reshape(1, 512)
    b1rep = jnp.tile(nn1_b1, 8).reshape(1, 512)
    b2r = jnp.pad(nn1_b2.reshape(_DIN, 8), ((0, 0), (0, 8)))
    oid = jnp.arange(512) // 64
    s1 = (oid[:, None] == jnp.arange(16)[None, :]).astype(jnp.float32)
    w22p = nn2_W2.reshape(64, 8, 8).transpose(0, 2, 1).reshape(64, 64)
    b22p = nn2_b2.reshape(8, 8).transpose().reshape(1, 64)
    iid = jnp.arange(64) % 8
    r2 = ((jnp.arange(16)[:, None] == iid[None, :])
          & (jnp.arange(16)[:, None] < 8)).astype(jnp.float32)
    oid2 = jnp.arange(64) // 8
    s2 = (oid2[:, None] == jnp.arange(16)[None, :]).astype(jnp.float32)
    zrows = jnp.zeros((_ZROWS, 16), jnp.float32)
    batch2d = batch.reshape(1, _N)

    xj = _sc_gather(x, src, _DIN)
    msg1 = _tc_edge1(ea, xj, w1rep, b1rep, a1, b2r, s1)
    parts1 = _sc_scatter(msg1, dst, zrows)
    h1pad, cnt = _tc_combine1(parts1, x, conv1_root_W,
                              conv1_bias.reshape(1, 8))
    h1j = _sc_gather(h1pad, src, 16)
    msg2 = _tc_edge2(ea, h1j, nn2_W1, nn2_b1.reshape(1, 64), w22p, b22p,
                     r2, s2)
    parts2 = _sc_scatter(msg2, dst, zrows)
    return _tc_final(parts2, h1pad, cnt, batch2d, conv2_root_W,
                     conv2_bias.reshape(1, 8), proj1_W,
                     proj1_b.reshape(1, 128), proj2_W, proj2_b.reshape(1, 4))
